# baseline (speedup 1.0000x reference)
import jax
import jax.numpy as jnp
from jax.experimental import pallas as pl

N = 4096
B = 4
IN_F = 256
OUT_F = 256

BK = 256


def _gcn_kernel(adj_ref, x_ref, w_ref, out_ref):
    k = pl.program_id(0)
    nk = pl.num_programs(0)

    @pl.when(k == 0)
    def _init():
        out_ref[...] = jnp.zeros_like(out_ref)

    adj_bf = adj_ref[...].astype(jnp.bfloat16)
    for b in range(B):
        out_ref[:, b, :] += jnp.dot(
            adj_bf,
            x_ref[:, b, :].astype(jnp.bfloat16),
            preferred_element_type=jnp.float32,
        )

    @pl.when(k == nk - 1)
    def _finish():
        w = w_ref[...].astype(jnp.bfloat16)
        for b in range(B):
            out_ref[:, b, :] = jnp.dot(
                out_ref[:, b, :].astype(jnp.bfloat16),
                w,
                preferred_element_type=jnp.float32,
            )


@jax.jit
def kernel(input, adj, weight):
    grid = (N // BK,)
    return pl.pallas_call(
        _gcn_kernel,
        grid=grid,
        in_specs=[
            pl.BlockSpec((N, BK), lambda k: (0, k)),
            pl.BlockSpec((BK, B, IN_F), lambda k: (k, 0, 0)),
            pl.BlockSpec((IN_F, OUT_F), lambda k: (0, 0)),
        ],
        out_specs=pl.BlockSpec((N, B, OUT_F), lambda k: (0, 0, 0)),
        out_shape=jax.ShapeDtypeStruct((N, B, OUT_F), jnp.float32),
    )(adj, input, weight)


# speedup vs baseline: 2.6392x; 2.6392x over previous
import jax
import jax.numpy as jnp
from jax.experimental import pallas as pl
from jax.experimental.pallas import tpu as pltpu

N = 4096
B = 4
IN_F = 256
OUT_F = 256

BM = 2048
BK = 512


def _gcn_kernel(adj_ref, x_ref, w_ref, out_ref, acc_ref):
    k = pl.program_id(1)
    nk = pl.num_programs(1)

    @pl.when(k == 0)
    def _init():
        acc_ref[...] = jnp.zeros_like(acc_ref)

    adj_bf = adj_ref[...].astype(jnp.bfloat16)
    for b in range(B):
        acc_ref[:, b * IN_F : (b + 1) * IN_F] += jnp.dot(
            adj_bf,
            x_ref[:, b, :].astype(jnp.bfloat16),
            preferred_element_type=jnp.float32,
        )

    @pl.when(k == nk - 1)
    def _finish():
        w = w_ref[...].astype(jnp.bfloat16)
        for b in range(B):
            out_ref[:, b, :] = jnp.dot(
                acc_ref[:, b * IN_F : (b + 1) * IN_F].astype(jnp.bfloat16),
                w,
                preferred_element_type=jnp.float32,
            )


@jax.jit
def kernel(input, adj, weight):
    grid = (N // BM, N // BK)
    return pl.pallas_call(
        _gcn_kernel,
        grid=grid,
        in_specs=[
            pl.BlockSpec((BM, BK), lambda m, k: (m, k)),
            pl.BlockSpec((BK, B, IN_F), lambda m, k: (k, 0, 0)),
            pl.BlockSpec((IN_F, OUT_F), lambda m, k: (0, 0)),
        ],
        out_specs=pl.BlockSpec((BM, B, OUT_F), lambda m, k: (m, 0, 0)),
        out_shape=jax.ShapeDtypeStruct((N, B, OUT_F), jnp.float32),
        scratch_shapes=[pltpu.VMEM((BM, B * IN_F), jnp.float32)],
    )(adj, input, weight)
